# baseline (speedup 1.0000x reference)
import functools

import jax
import jax.numpy as jnp
from jax import lax
from jax.experimental import pallas as pl
from jax.experimental.pallas import tpu as pltpu
from jax.experimental.pallas import tpu_sc as plsc

E = 8
K = 2
H = 2048
I = 768
T = 2048
PAIRS = T * K

BLK = 256
NBLK = PAIRS // BLK + E
PAD_T = NBLK * BLK

NC = 2
NS = 16
L = 16
NW = NC * NS
SLOTS_W = PAD_T // NW
TOK_W = T // NW
GCH = 8
NSTREAM = 4
CCH = 8


def _router_plan_body(x_ref, gw_ref, pos_ref, w_ref, bexp_ref, meta_ref):
    f32, i32 = jnp.float32, jnp.int32
    logits = jnp.dot(x_ref[...], gw_ref[...], preferred_element_type=f32)

    iota_e = lax.broadcasted_iota(i32, (T, E), 1)
    m1 = jnp.max(logits, axis=1, keepdims=True)
    a1 = jnp.min(jnp.where(logits == m1, iota_e, E), axis=1, keepdims=True)
    l2 = jnp.where(iota_e == a1, -jnp.inf, logits)
    m2 = jnp.max(l2, axis=1, keepdims=True)
    a2 = jnp.min(jnp.where(l2 == m2, iota_e, E), axis=1, keepdims=True)

    e2 = jnp.exp(m2 - m1)
    w1 = 1.0 / (1.0 + e2)
    w2 = e2 / (1.0 + e2)
    w_ref[...] = jnp.concatenate([w1, w2], axis=1)

    onehot0 = (iota_e == a1).astype(f32)
    onehot1 = (iota_e == a2).astype(f32)

    r_io = lax.broadcasted_iota(i32, (T, T), 0)
    c_io = lax.broadcasted_iota(i32, (T, T), 1)
    tri = (c_io <= r_io).astype(f32)
    c0 = jnp.dot(tri, onehot0, preferred_element_type=f32)
    c1 = jnp.dot(tri, onehot1, preferred_element_type=f32)
    c0i = c0.astype(i32)
    c1i = c1.astype(i32)
    cnt0 = c0i[T - 1:T, :]
    cnt1 = c1i[T - 1:T, :]
    cnt = cnt0 + cnt1
    nblk = (cnt + (BLK - 1)) // BLK
    meta_ref[...] = jnp.sum(nblk, axis=1, keepdims=True)

    er = lax.broadcasted_iota(i32, (E, E), 0)
    ec = lax.broadcasted_iota(i32, (E, E), 1)
    boff = jnp.sum(jnp.where(er < ec, jnp.broadcast_to(nblk.reshape(E, 1), (E, E)), 0),
                   axis=0, keepdims=True)
    padoff = boff * BLK

    mask0 = onehot0 > 0
    mask1 = onehot1 > 0
    pad0 = jnp.sum(jnp.where(mask0, padoff, 0), axis=1, keepdims=True)
    rank0 = jnp.sum(jnp.where(mask0, c0i, 0), axis=1, keepdims=True)
    pos0 = pad0 + rank0 - 1
    pad1 = jnp.sum(jnp.where(mask1, padoff, 0), axis=1, keepdims=True)
    base1 = jnp.sum(jnp.where(mask1, cnt0, 0), axis=1, keepdims=True)
    rank1 = jnp.sum(jnp.where(mask1, c1i, 0), axis=1, keepdims=True)
    pos1 = pad1 + base1 + rank1 - 1
    pos_ref[...] = jnp.concatenate([pos0, pos1], axis=1)

    b_io = lax.broadcasted_iota(i32, (NBLK, E), 0)
    ge = (b_io >= boff).astype(i32)
    bexp_ref[...] = jnp.sum(ge, axis=1, keepdims=True) - 1


def _router_plan(x, gate_w):
    return pl.pallas_call(
        _router_plan_body,
        out_shape=[
            jax.ShapeDtypeStruct((T, K), jnp.int32),
            jax.ShapeDtypeStruct((T, K), jnp.float32),
            jax.ShapeDtypeStruct((NBLK, 1), jnp.int32),
            jax.ShapeDtypeStruct((1, 1), jnp.int32),
        ],
    )(x, gate_w)


def _dispatch_body(x_hbm, pos_hbm, wp_hbm, xs_hbm, ws_hbm,
                   pos_v, wp_v, st_v, ws_v, rows0, rows1, rows2, rows3,
                   gs0, gs1, gs2, gs3, ws0, ws1, ws2, ws3):
    wid = lax.axis_index("s") * NC + lax.axis_index("c")
    pltpu.sync_copy(pos_hbm, pos_v)
    pltpu.sync_copy(wp_hbm, wp_v)

    zi = jnp.zeros((L,), jnp.int32)
    zf = jnp.zeros((L,), jnp.float32)

    def mz(i, _):
        for u in range(4):
            st_v[pl.ds((i * 4 + u) * L, L)] = zi
            ws_v[pl.ds((i * 4 + u) * L, L)] = zf
        return 0
    lax.fori_loop(0, PAD_T // L // 4, mz, 0)

    lane = lax.iota(jnp.int32, L)

    def sc(i, _):
        for u in range(4):
            o = (i * 4 + u) * L
            idx = pos_v[pl.ds(o, L)]
            plsc.store_scatter(st_v, [idx], (lane + o) & (T - 1))
            plsc.store_scatter(ws_v, [idx], wp_v[pl.ds(o, L)])
        return 0
    lax.fori_loop(0, PAIRS // L // 4, sc, 0)

    base = wid * SLOTS_W
    pltpu.sync_copy(ws_v.at[pl.ds(base, SLOTS_W)], ws_hbm.at[pl.ds(base, SLOTS_W)])

    nchunk = SLOTS_W // GCH
    bufs = (rows0, rows1, rows2, rows3)
    gsems = (gs0, gs1, gs2, gs3)
    wsems = (ws0, ws1, ws2, ws3)
    gdesc = [None] * NSTREAM
    wdesc = [None] * NSTREAM

    def fire_gather(c):
        p = c % NSTREAM
        s = base + c * GCH
        gdesc[p] = pltpu.async_copy(x_hbm.at[st_v.at[pl.ds(s, GCH)]],
                                    bufs[p], gsems[p])

    for c in range(NSTREAM - 1):
        fire_gather(c)
    for c in range(nchunk):
        p = c % NSTREAM
        nxt = c + NSTREAM - 1
        if nxt < nchunk:
            q = nxt % NSTREAM
            if wdesc[q] is not None:
                wdesc[q].wait()
            fire_gather(nxt)
        gdesc[p].wait()
        wdesc[p] = pltpu.async_copy(bufs[p],
                                    xs_hbm.at[pl.ds(base + c * GCH, GCH)],
                                    wsems[p])
    for p in range(NSTREAM):
        if wdesc[p] is not None:
            wdesc[p].wait()


def _dispatch(x, pos_flat, wp_flat):
    mesh = plsc.VectorSubcoreMesh(core_axis_name="c", subcore_axis_name="s",
                                  num_cores=NC, num_subcores=NS)
    kern = pl.kernel(
        _dispatch_body,
        out_type=[
            jax.ShapeDtypeStruct((PAD_T, H), jnp.float32),
            jax.ShapeDtypeStruct((PAD_T,), jnp.float32),
        ],
        mesh=mesh,
        scratch_types=[
            pltpu.VMEM((PAIRS,), jnp.int32),
            pltpu.VMEM((PAIRS,), jnp.float32),
            pltpu.VMEM((PAD_T,), jnp.int32),
            pltpu.VMEM((PAD_T,), jnp.float32),
            pltpu.VMEM((GCH, H), jnp.float32),
            pltpu.VMEM((GCH, H), jnp.float32),
            pltpu.VMEM((GCH, H), jnp.float32),
            pltpu.VMEM((GCH, H), jnp.float32),
            pltpu.SemaphoreType.DMA,
            pltpu.SemaphoreType.DMA,
            pltpu.SemaphoreType.DMA,
            pltpu.SemaphoreType.DMA,
            pltpu.SemaphoreType.DMA,
            pltpu.SemaphoreType.DMA,
            pltpu.SemaphoreType.DMA,
            pltpu.SemaphoreType.DMA,
        ],
        compiler_params=pltpu.CompilerParams(needs_layout_passes=False),
    )
    return kern(x, pos_flat, wp_flat)


def _ffn_body(bexp_ref, meta_ref, xs_ref, wg_ref, wu_ref, wd_ref, ws_ref, out_ref):
    b = pl.program_id(0)

    @pl.when(b < meta_ref[0])
    def _():
        xb = xs_ref[...]
        g = jnp.dot(xb, wg_ref[0], preferred_element_type=jnp.float32,
                    precision=lax.Precision.DEFAULT)
        u = jnp.dot(xb, wu_ref[0], preferred_element_type=jnp.float32,
                    precision=lax.Precision.DEFAULT)
        h = g * (1.0 / (1.0 + jnp.exp(-g))) * u
        d = jnp.dot(h, wd_ref[0], preferred_element_type=jnp.float32,
                    precision=lax.Precision.DEFAULT)
        out_ref[...] = d * ws_ref[...]


def _expert_ffn(bexp, meta, xs, gate_proj_w, up_proj_w, down_proj_w, wsort):
    grid_spec = pltpu.PrefetchScalarGridSpec(
        num_scalar_prefetch=2,
        grid=(NBLK,),
        in_specs=[
            pl.BlockSpec((BLK, H), lambda b, be, mt: (jnp.minimum(b, mt[0] - 1), 0)),
            pl.BlockSpec((1, H, I), lambda b, be, mt: (be[b], 0, 0)),
            pl.BlockSpec((1, H, I), lambda b, be, mt: (be[b], 0, 0)),
            pl.BlockSpec((1, I, H), lambda b, be, mt: (be[b], 0, 0)),
            pl.BlockSpec((BLK, 1), lambda b, be, mt: (jnp.minimum(b, mt[0] - 1), 0)),
        ],
        out_specs=pl.BlockSpec((BLK, H),
                               lambda b, be, mt: (jnp.minimum(b, mt[0] - 1), 0)),
    )
    return pl.pallas_call(
        _ffn_body,
        grid_spec=grid_spec,
        out_shape=jax.ShapeDtypeStruct((PAD_T, H), jnp.float32),
        compiler_params=pltpu.CompilerParams(
            dimension_semantics=("arbitrary",)),
    )(bexp, meta, xs, gate_proj_w, up_proj_w, down_proj_w, wsort)


def _combine_body(ys_hbm, pos0_hbm, pos1_hbm, out_hbm, p0_v, p1_v,
                  r0a, r1a, r0b, r1b, gsa, gsb, wsa, wsb):
    wid = lax.axis_index("s") * NC + lax.axis_index("c")
    tb = wid * TOK_W
    pltpu.sync_copy(pos0_hbm.at[pl.ds(tb, TOK_W)], p0_v)
    pltpu.sync_copy(pos1_hbm.at[pl.ds(tb, TOK_W)], p1_v)

    nchunk = TOK_W // CCH
    r0s = (r0a, r0b)
    r1s = (r1a, r1b)
    gsems = (gsa, gsb)
    wsems = (wsa, wsb)
    gd = [None, None]
    wd = [None, None]

    def fire_gathers(c):
        p = c % 2
        s = pl.ds(c * CCH, CCH)
        d0 = pltpu.async_copy(ys_hbm.at[p0_v.at[s]], r0s[p], gsems[p])
        d1 = pltpu.async_copy(ys_hbm.at[p1_v.at[s]], r1s[p], gsems[p])
        gd[p] = (d0, d1)

    fire_gathers(0)
    for c in range(nchunk):
        p = c % 2
        if c + 1 < nchunk:
            if wd[1 - p] is not None:
                wd[1 - p].wait()
            fire_gathers(c + 1)
        gd[p][0].wait()
        gd[p][1].wait()
        r0, r1 = r0s[p], r1s[p]
        for row in range(CCH):
            def inner(j, _, row=row):
                s = pl.ds(j * 4 * L, L)
                s1 = pl.ds((j * 4 + 1) * L, L)
                s2 = pl.ds((j * 4 + 2) * L, L)
                s3 = pl.ds((j * 4 + 3) * L, L)
                r0[row, s] = r0[row, s] + r1[row, s]
                r0[row, s1] = r0[row, s1] + r1[row, s1]
                r0[row, s2] = r0[row, s2] + r1[row, s2]
                r0[row, s3] = r0[row, s3] + r1[row, s3]
                return 0
            lax.fori_loop(0, H // L // 4, inner, 0)
        wd[p] = pltpu.async_copy(r0, out_hbm.at[pl.ds(tb + c * CCH, CCH)],
                                 wsems[p])
    for p in range(2):
        if wd[p] is not None:
            wd[p].wait()


def _combine(ysw, pos0, pos1):
    mesh = plsc.VectorSubcoreMesh(core_axis_name="c", subcore_axis_name="s",
                                  num_cores=NC, num_subcores=NS)
    kern = pl.kernel(
        _combine_body,
        out_type=jax.ShapeDtypeStruct((T, H), jnp.float32),
        mesh=mesh,
        scratch_types=[
            pltpu.VMEM((TOK_W,), jnp.int32),
            pltpu.VMEM((TOK_W,), jnp.int32),
            pltpu.VMEM((CCH, H), jnp.float32),
            pltpu.VMEM((CCH, H), jnp.float32),
            pltpu.VMEM((CCH, H), jnp.float32),
            pltpu.VMEM((CCH, H), jnp.float32),
            pltpu.SemaphoreType.DMA,
            pltpu.SemaphoreType.DMA,
            pltpu.SemaphoreType.DMA,
            pltpu.SemaphoreType.DMA,
        ],
        compiler_params=pltpu.CompilerParams(needs_layout_passes=False),
    )
    return kern(ysw, pos0, pos1)


def kernel(hidden_states, gate_w, gate_proj_w, up_proj_w, down_proj_w):
    B, S, Hh = hidden_states.shape
    x = hidden_states.reshape(S * B, Hh)

    pos2, w2, bexp2, meta2 = _router_plan(x, gate_w)

    pos_flat = jnp.concatenate([pos2[:, 0], pos2[:, 1]])
    wp_flat = jnp.concatenate([w2[:, 0], w2[:, 1]])

    xs, wsort = _dispatch(x, pos_flat, wp_flat)

    ysw = _expert_ffn(bexp2.reshape(NBLK), meta2.reshape(1), xs,
                      gate_proj_w, up_proj_w, down_proj_w,
                      wsort.reshape(PAD_T, 1))

    out = _combine(ysw, pos2[:, 0], pos2[:, 1])
    return out.reshape(B, S, Hh)


# speedup vs baseline: 1.9132x; 1.9132x over previous
import functools

import jax
import jax.numpy as jnp
from jax import lax
from jax.experimental import pallas as pl
from jax.experimental.pallas import tpu as pltpu
from jax.experimental.pallas import tpu_sc as plsc

E = 8
K = 2
H = 2048
I = 768
T = 2048
PAIRS = T * K

BLK = 256
NBLK = PAIRS // BLK + E
PAD_T = NBLK * BLK

NC = 2
NS = 16
L = 16
NW = NC * NS
SLOTS_W = PAD_T // NW
TOK_W = T // NW
XCH = 16
CCH = 8


def _router_plan_body(x_ref, gw_ref, pos_ref, w_ref, bexp_ref, meta_ref):
    f32, i32 = jnp.float32, jnp.int32
    logits = jnp.dot(x_ref[...], gw_ref[...], preferred_element_type=f32)

    iota_e = lax.broadcasted_iota(i32, (T, E), 1)
    m1 = jnp.max(logits, axis=1, keepdims=True)
    a1 = jnp.min(jnp.where(logits == m1, iota_e, E), axis=1, keepdims=True)
    l2 = jnp.where(iota_e == a1, -jnp.inf, logits)
    m2 = jnp.max(l2, axis=1, keepdims=True)
    a2 = jnp.min(jnp.where(l2 == m2, iota_e, E), axis=1, keepdims=True)

    e2 = jnp.exp(m2 - m1)
    w1 = 1.0 / (1.0 + e2)
    w2 = e2 / (1.0 + e2)
    w_ref[...] = jnp.concatenate([w1, w2], axis=1)

    onehot0 = (iota_e == a1).astype(f32)
    onehot1 = (iota_e == a2).astype(f32)

    r_io = lax.broadcasted_iota(i32, (T, T), 0)
    c_io = lax.broadcasted_iota(i32, (T, T), 1)
    tri = (c_io <= r_io).astype(f32)
    c0 = jnp.dot(tri, onehot0, preferred_element_type=f32)
    c1 = jnp.dot(tri, onehot1, preferred_element_type=f32)
    c0i = c0.astype(i32)
    c1i = c1.astype(i32)
    cnt0 = c0i[T - 1:T, :]
    cnt1 = c1i[T - 1:T, :]
    cnt = cnt0 + cnt1
    nblk = (cnt + (BLK - 1)) // BLK
    meta_ref[...] = jnp.sum(nblk, axis=1, keepdims=True)

    er = lax.broadcasted_iota(i32, (E, E), 0)
    ec = lax.broadcasted_iota(i32, (E, E), 1)
    boff = jnp.sum(jnp.where(er < ec, jnp.broadcast_to(nblk.reshape(E, 1), (E, E)), 0),
                   axis=0, keepdims=True)
    padoff = boff * BLK

    mask0 = onehot0 > 0
    mask1 = onehot1 > 0
    pad0 = jnp.sum(jnp.where(mask0, padoff, 0), axis=1, keepdims=True)
    rank0 = jnp.sum(jnp.where(mask0, c0i, 0), axis=1, keepdims=True)
    pos0 = pad0 + rank0 - 1
    pad1 = jnp.sum(jnp.where(mask1, padoff, 0), axis=1, keepdims=True)
    base1 = jnp.sum(jnp.where(mask1, cnt0, 0), axis=1, keepdims=True)
    rank1 = jnp.sum(jnp.where(mask1, c1i, 0), axis=1, keepdims=True)
    pos1 = pad1 + base1 + rank1 - 1
    pos_ref[...] = jnp.concatenate([pos0, pos1], axis=1)

    b_io = lax.broadcasted_iota(i32, (NBLK, E), 0)
    ge = (b_io >= boff).astype(i32)
    bexp_ref[...] = jnp.sum(ge, axis=1, keepdims=True) - 1


def _router_plan(x, gate_w):
    return pl.pallas_call(
        _router_plan_body,
        out_shape=[
            jax.ShapeDtypeStruct((T, K), jnp.int32),
            jax.ShapeDtypeStruct((T, K), jnp.float32),
            jax.ShapeDtypeStruct((NBLK, 1), jnp.int32),
            jax.ShapeDtypeStruct((1, 1), jnp.int32),
        ],
    )(x, gate_w)


def _dispatch_body(x_hbm, posi_hbm, xs_hbm,
                   pv, idx_v, buf0, buf1, ls0, ls1, ss0, ss1):
    wid = lax.axis_index("s") * NC + lax.axis_index("c")
    tb = wid * TOK_W
    pltpu.sync_copy(posi_hbm.at[pl.ds(tb * K, K * TOK_W)], pv)

    lane = lax.iota(jnp.int32, L)
    nch = TOK_W // XCH
    for j in range(nch):
        idx_v[2 * j, :] = plsc.load_gather(pv, [j * 2 * XCH + 2 * lane])
        idx_v[2 * j + 1, :] = plsc.load_gather(pv, [j * 2 * XCH + 2 * lane + 1])

    bufs = (buf0, buf1)
    lsems = (ls0, ls1)
    ssems = (ss0, ss1)
    ld = [None, None]
    sc = [[None, None], [None, None]]

    def fire_load(j):
        p = j % 2
        ld[p] = pltpu.async_copy(x_hbm.at[pl.ds(tb + j * XCH, XCH)],
                                 bufs[p], lsems[p])

    fire_load(0)
    for j in range(nch):
        p = j % 2
        if j + 1 < nch:
            q = (j + 1) % 2
            for d in sc[q]:
                if d is not None:
                    d.wait()
            sc[q] = [None, None]
            fire_load(j + 1)
        ld[p].wait()
        sc[p][0] = pltpu.async_copy(bufs[p], xs_hbm.at[idx_v.at[2 * j]], ssems[p])
        sc[p][1] = pltpu.async_copy(bufs[p], xs_hbm.at[idx_v.at[2 * j + 1]], ssems[p])
    for p in range(2):
        for d in sc[p]:
            if d is not None:
                d.wait()


def _dispatch(x, posi):
    mesh = plsc.VectorSubcoreMesh(core_axis_name="c", subcore_axis_name="s",
                                  num_cores=NC, num_subcores=NS)
    kern = pl.kernel(
        _dispatch_body,
        out_type=jax.ShapeDtypeStruct((PAD_T, H), x.dtype),
        mesh=mesh,
        scratch_types=[
            pltpu.VMEM((K * TOK_W,), jnp.int32),
            pltpu.VMEM((2 * TOK_W // XCH, L), jnp.int32),
            pltpu.VMEM((XCH, H), x.dtype),
            pltpu.VMEM((XCH, H), x.dtype),
            pltpu.SemaphoreType.DMA,
            pltpu.SemaphoreType.DMA,
            pltpu.SemaphoreType.DMA,
            pltpu.SemaphoreType.DMA,
        ],
        compiler_params=pltpu.CompilerParams(needs_layout_passes=False),
    )
    return kern(x, posi)


def _ffn_body(bexp_ref, meta_ref, xs_ref, wg_ref, wu_ref, wd_ref, out_ref):
    b = pl.program_id(0)

    @pl.when(b < meta_ref[0])
    def _():
        xb = xs_ref[...]
        g = jnp.dot(xb, wg_ref[0], preferred_element_type=jnp.float32,
                    precision=lax.Precision.DEFAULT)
        u = jnp.dot(xb, wu_ref[0], preferred_element_type=jnp.float32,
                    precision=lax.Precision.DEFAULT)
        h = g * (1.0 / (1.0 + jnp.exp(-g))) * u
        out_ref[...] = jnp.dot(h, wd_ref[0], preferred_element_type=jnp.float32,
                               precision=lax.Precision.DEFAULT)


def _expert_ffn(bexp, meta, xs, gate_proj_w, up_proj_w, down_proj_w):
    grid_spec = pltpu.PrefetchScalarGridSpec(
        num_scalar_prefetch=2,
        grid=(NBLK,),
        in_specs=[
            pl.BlockSpec((BLK, H), lambda b, be, mt: (jnp.minimum(b, mt[0] - 1), 0)),
            pl.BlockSpec((1, H, I), lambda b, be, mt: (be[b], 0, 0)),
            pl.BlockSpec((1, H, I), lambda b, be, mt: (be[b], 0, 0)),
            pl.BlockSpec((1, I, H), lambda b, be, mt: (be[b], 0, 0)),
        ],
        out_specs=pl.BlockSpec((BLK, H),
                               lambda b, be, mt: (jnp.minimum(b, mt[0] - 1), 0)),
    )
    return pl.pallas_call(
        _ffn_body,
        grid_spec=grid_spec,
        out_shape=jax.ShapeDtypeStruct((PAD_T, H), jnp.float32),
        compiler_params=pltpu.CompilerParams(
            dimension_semantics=("arbitrary",)),
    )(bexp, meta, xs, gate_proj_w, up_proj_w, down_proj_w)


def _combine_body(ys_hbm, posi_hbm, wpi_hbm, out_hbm, pv, wv, p0_v, p1_v,
                  r0a, r1a, r0b, r1b, gsa, gsb, wsa, wsb):
    wid = lax.axis_index("s") * NC + lax.axis_index("c")
    tb = wid * TOK_W
    pltpu.sync_copy(posi_hbm.at[pl.ds(tb * K, K * TOK_W)], pv)
    pltpu.sync_copy(wpi_hbm.at[pl.ds(tb * K, K * TOK_W)], wv)

    lane = lax.iota(jnp.int32, L)
    for j in range(TOK_W // L):
        p0_v[pl.ds(j * L, L)] = plsc.load_gather(pv, [j * 2 * L + 2 * lane])
        p1_v[pl.ds(j * L, L)] = plsc.load_gather(pv, [j * 2 * L + 2 * lane + 1])

    nchunk = TOK_W // CCH
    r0s = (r0a, r0b)
    r1s = (r1a, r1b)
    gsems = (gsa, gsb)
    wsems = (wsa, wsb)
    gd = [None, None]
    wd = [None, None]

    def fire_gathers(c):
        p = c % 2
        s = pl.ds(c * CCH, CCH)
        d0 = pltpu.async_copy(ys_hbm.at[p0_v.at[s]], r0s[p], gsems[p])
        d1 = pltpu.async_copy(ys_hbm.at[p1_v.at[s]], r1s[p], gsems[p])
        gd[p] = (d0, d1)

    fire_gathers(0)
    for c in range(nchunk):
        p = c % 2
        if c + 1 < nchunk:
            if wd[1 - p] is not None:
                wd[1 - p].wait()
            fire_gathers(c + 1)
        gd[p][0].wait()
        gd[p][1].wait()
        r0, r1 = r0s[p], r1s[p]
        wc = wv[pl.ds(c * 2 * CCH, L)]
        for row in range(CCH):
            w0 = wc.at[jnp.full((L,), 2 * row, jnp.int32)].get(
                mode="promise_in_bounds")
            w1 = wc.at[jnp.full((L,), 2 * row + 1, jnp.int32)].get(
                mode="promise_in_bounds")

            def inner(j, _, row=row, w0=w0, w1=w1):
                s = pl.ds(j * 4 * L, L)
                s1 = pl.ds((j * 4 + 1) * L, L)
                s2 = pl.ds((j * 4 + 2) * L, L)
                s3 = pl.ds((j * 4 + 3) * L, L)
                r0[row, s] = w0 * r0[row, s] + w1 * r1[row, s]
                r0[row, s1] = w0 * r0[row, s1] + w1 * r1[row, s1]
                r0[row, s2] = w0 * r0[row, s2] + w1 * r1[row, s2]
                r0[row, s3] = w0 * r0[row, s3] + w1 * r1[row, s3]
                return 0
            lax.fori_loop(0, H // L // 4, inner, 0)
        wd[p] = pltpu.async_copy(r0, out_hbm.at[pl.ds(tb + c * CCH, CCH)],
                                 wsems[p])
    for p in range(2):
        if wd[p] is not None:
            wd[p].wait()


def _combine(ysw, posi, wpi):
    mesh = plsc.VectorSubcoreMesh(core_axis_name="c", subcore_axis_name="s",
                                  num_cores=NC, num_subcores=NS)
    kern = pl.kernel(
        _combine_body,
        out_type=jax.ShapeDtypeStruct((T, H), jnp.float32),
        mesh=mesh,
        scratch_types=[
            pltpu.VMEM((K * TOK_W,), jnp.int32),
            pltpu.VMEM((K * TOK_W,), jnp.float32),
            pltpu.VMEM((TOK_W,), jnp.int32),
            pltpu.VMEM((TOK_W,), jnp.int32),
            pltpu.VMEM((CCH, H), jnp.float32),
            pltpu.VMEM((CCH, H), jnp.float32),
            pltpu.VMEM((CCH, H), jnp.float32),
            pltpu.VMEM((CCH, H), jnp.float32),
            pltpu.SemaphoreType.DMA,
            pltpu.SemaphoreType.DMA,
            pltpu.SemaphoreType.DMA,
            pltpu.SemaphoreType.DMA,
        ],
        compiler_params=pltpu.CompilerParams(needs_layout_passes=False),
    )
    return kern(ysw, posi, wpi)


def kernel(hidden_states, gate_w, gate_proj_w, up_proj_w, down_proj_w):
    B, S, Hh = hidden_states.shape
    x = hidden_states.reshape(S * B, Hh)

    pos2, w2, bexp2, meta2 = _router_plan(x, gate_w)

    posi = pos2.reshape(PAIRS)
    wpi = w2.reshape(PAIRS)

    xs = _dispatch(x, posi)

    ysw = _expert_ffn(bexp2.reshape(NBLK), meta2.reshape(1), xs,
                      gate_proj_w, up_proj_w, down_proj_w)

    out = _combine(ysw, posi, wpi)
    return out.reshape(B, S, Hh)
